# baseline (speedup 1.0000x reference)
import functools

import jax
import jax.numpy as jnp
import numpy as np
from jax import lax
from jax.experimental import pallas as pl
from jax.experimental.pallas import tpu as pltpu
from jax.experimental.pallas import tpu_sc as plsc

_HIDDEN = 128
_ENT_DIM = 256
_B = 1024
_NNEG = 256
_GAMMA = 12.0
_EMB_RANGE = (12.0 + 2.0) / _HIDDEN
_PHASE_K = float(np.pi) / _EMB_RANGE

_TILES = 32
_NCALLS = 2
_B_CALL = _B // _NCALLS
_B_PER_TILE = _B_CALL // _TILES
_ROWS_PER_TILE = _B_PER_TILE * _NNEG
_CHUNK = 128
_PAIRS = _ROWS_PER_TILE // (2 * _CHUNK)
_CROWS = _B_CALL * _NNEG

_SIN_C = (9.9999999443e-01, -1.6666664567e-01, 8.3333102843e-03,
          -1.9840151690e-04, 2.7529392628e-06, -2.4676469125e-08,
          1.3449911084e-10)
_COS_C = (1.0000000001e+00, -4.9999999854e-01, 4.1666663479e-02,
          -1.3888863033e-03, 2.4800553772e-05, -2.7534807478e-07,
          2.0603622903e-09, -9.7225822060e-12)


def _sin_poly(t):
    t2 = t * t
    r = jnp.float32(_SIN_C[-1])
    for c in _SIN_C[-2::-1]:
        r = r * t2 + jnp.float32(c)
    return r * t


def _cos_poly(t):
    t2 = t * t
    r = jnp.float32(_COS_C[-1])
    for c in _COS_C[-2::-1]:
        r = r * t2 + jnp.float32(c)
    return r


def _sc_body(head_idx_h, rel_idx_h, tail_idx_h, ent_h, relemb_h, s_h,
             hidx_v, ridx_v, idx_v, head_v, rel_v, rot_v,
             buf0, buf1, sbuf0, sbuf1, sem0, sem_a, sem_b, sem_oa, sem_ob):
    wid = lax.axis_index("s") * 2 + lax.axis_index("c")
    tb = wid * _B_PER_TILE
    row0 = tb * _NNEG

    pltpu.sync_copy(head_idx_h.at[pl.ds(tb, _B_PER_TILE)], hidx_v)
    pltpu.sync_copy(rel_idx_h.at[pl.ds(tb, _B_PER_TILE)], ridx_v)
    pltpu.sync_copy(tail_idx_h.at[pl.ds(row0, _ROWS_PER_TILE)], idx_v)

    def start_gather(c, buf, sem):
        pltpu.make_async_copy(
            ent_h.at[idx_v.at[pl.ds(pl.multiple_of(c * _CHUNK, _CHUNK),
                                    _CHUNK)]],
            buf, sem).start()

    def wait_gather(buf, sem):
        pltpu.make_async_copy(
            ent_h.at[idx_v.at[pl.ds(0, _CHUNK)]], buf, sem).wait()

    def start_out(c, sbuf, sem):
        pltpu.make_async_copy(
            sbuf, s_h.at[pl.ds(row0 + c * _CHUNK, _CHUNK)], sem).start()

    def wait_out(sbuf, sem):
        pltpu.make_async_copy(
            sbuf, s_h.at[pl.ds(row0, _CHUNK)], sem).wait()

    start_gather(0, buf0, sem_a)
    start_gather(1, buf1, sem_b)

    pltpu.async_copy(ent_h.at[hidx_v], head_v, sem0).wait()
    pltpu.async_copy(relemb_h.at[ridx_v], rel_v, sem0).wait()

    def rot_body(b, carry):
        for hv in range(_HIDDEN // 16):
            sl = pl.ds(hv * 16, 16)
            ph = rel_v[b, sl] * jnp.float32(_PHASE_K)
            cr = _cos_poly(ph)
            sr = _sin_poly(ph)
            rh = head_v[b, sl]
            ih = head_v[b, pl.ds(_HIDDEN + hv * 16, 16)]
            rot_v[b, sl] = rh * cr - ih * sr
            rot_v[b, pl.ds(_HIDDEN + hv * 16, 16)] = rh * sr + ih * cr
        return carry

    lax.fori_loop(0, _B_PER_TILE, rot_body, 0)

    def compute_chunk(buf, sbuf, b):
        rotr = [rot_v[b, pl.ds(hv * 16, 16)]
                for hv in range(_HIDDEN // 16)]
        roti = [rot_v[b, pl.ds(_HIDDEN + hv * 16, 16)]
                for hv in range(_HIDDEN // 16)]

        def row_body(j, carry):
            for hv in range(_HIDDEN // 16):
                rt = buf[j, pl.ds(hv * 16, 16)]
                it = buf[j, pl.ds(_HIDDEN + hv * 16, 16)]
                d1 = rotr[hv] - rt
                d2 = roti[hv] - it
                sbuf[j, pl.ds(hv * 16, 16)] = d1 * d1 + d2 * d2
            return carry

        lax.fori_loop(0, _CHUNK, row_body, 0)

    def pair_body(i, carry):
        wait_gather(buf0, sem_a)

        @pl.when(i > 0)
        def _():
            wait_out(sbuf0, sem_oa)

        compute_chunk(buf0, sbuf0, i)

        @pl.when(i < _PAIRS - 1)
        def _():
            start_gather(2 * i + 2, buf0, sem_a)

        start_out(2 * i, sbuf0, sem_oa)

        wait_gather(buf1, sem_b)

        @pl.when(i > 0)
        def _():
            wait_out(sbuf1, sem_ob)

        compute_chunk(buf1, sbuf1, i)

        @pl.when(i < _PAIRS - 1)
        def _():
            start_gather(2 * i + 3, buf1, sem_b)

        start_out(2 * i + 1, sbuf1, sem_ob)
        return carry

    lax.fori_loop(0, _PAIRS, pair_body, 0)
    wait_out(sbuf0, sem_oa)
    wait_out(sbuf1, sem_ob)


@functools.lru_cache(maxsize=1)
def _build_sq():
    return functools.partial(
        pl.kernel,
        out_type=jax.ShapeDtypeStruct((_CROWS, _HIDDEN), jnp.float32),
        scratch_types=[
            pltpu.VMEM((_B_PER_TILE,), jnp.int32),
            pltpu.VMEM((_B_PER_TILE,), jnp.int32),
            pltpu.VMEM((_ROWS_PER_TILE,), jnp.int32),
            pltpu.VMEM((_B_PER_TILE, _ENT_DIM), jnp.float32),
            pltpu.VMEM((_B_PER_TILE, _HIDDEN), jnp.float32),
            pltpu.VMEM((_B_PER_TILE, _ENT_DIM), jnp.float32),
            pltpu.VMEM((_CHUNK, _ENT_DIM), jnp.float32),
            pltpu.VMEM((_CHUNK, _ENT_DIM), jnp.float32),
            pltpu.VMEM((_CHUNK, _HIDDEN), jnp.float32),
            pltpu.VMEM((_CHUNK, _HIDDEN), jnp.float32),
            pltpu.SemaphoreType.DMA,
            pltpu.SemaphoreType.DMA,
            pltpu.SemaphoreType.DMA,
            pltpu.SemaphoreType.DMA,
            pltpu.SemaphoreType.DMA,
        ],
        mesh=plsc.VectorSubcoreMesh(core_axis_name="c", subcore_axis_name="s"),
    )(_sc_body)


_TC_ROWS = 8


def _tc_score_body(s_ref, out_ref):
    s = s_ref[...]
    v = jnp.sqrt(s).reshape(_TC_ROWS, _NNEG, _HIDDEN)
    out_ref[...] = jnp.float32(_GAMMA) - jnp.sum(v, axis=-1)


@functools.lru_cache(maxsize=1)
def _build_score():
    grid = _B_CALL // _TC_ROWS
    return pl.pallas_call(
        _tc_score_body,
        grid=(grid,),
        in_specs=[
            pl.BlockSpec((_TC_ROWS * _NNEG, _HIDDEN), lambda i: (i, 0)),
        ],
        out_specs=pl.BlockSpec((_TC_ROWS, _NNEG), lambda i: (i, 0)),
        out_shape=jax.ShapeDtypeStruct((_B_CALL, _NNEG), jnp.float32),
    )


@jax.jit
def kernel(head_idx, rel_idx, neg_tail_idx, entity_embedding,
           relation_embedding):
    tail_flat = neg_tail_idx.reshape(-1)
    sq = _build_sq()
    score = _build_score()
    parts = []
    for k in range(_NCALLS):
        b0 = k * _B_CALL
        s = sq(head_idx[b0:b0 + _B_CALL],
               rel_idx[b0:b0 + _B_CALL],
               tail_flat[b0 * _NNEG:(b0 + _B_CALL) * _NNEG],
               entity_embedding, relation_embedding)
        parts.append(score(s))
    return jnp.concatenate(parts, axis=0)


# speedup vs baseline: 1.7354x; 1.7354x over previous
import functools

import jax
import jax.numpy as jnp
import numpy as np
from jax import lax
from jax.experimental import pallas as pl
from jax.experimental.pallas import tpu as pltpu
from jax.experimental.pallas import tpu_sc as plsc

_HIDDEN = 128
_ENT_DIM = 256
_B = 1024
_NNEG = 256
_GAMMA = 12.0
_EMB_RANGE = (12.0 + 2.0) / _HIDDEN
_PHASE_K = float(np.pi) / _EMB_RANGE

_TILES = 32
_NCALLS = 2
_B_CALL = _B // _NCALLS
_B_PER_TILE = _B_CALL // _TILES
_ROWS_PER_TILE = _B_PER_TILE * _NNEG
_CHUNK = 128
_PAIRS = _ROWS_PER_TILE // (2 * _CHUNK)
_CROWS = _B_CALL * _NNEG

_SIN_C = (9.9999999443e-01, -1.6666664567e-01, 8.3333102843e-03,
          -1.9840151690e-04, 2.7529392628e-06, -2.4676469125e-08,
          1.3449911084e-10)
_COS_C = (1.0000000001e+00, -4.9999999854e-01, 4.1666663479e-02,
          -1.3888863033e-03, 2.4800553772e-05, -2.7534807478e-07,
          2.0603622903e-09, -9.7225822060e-12)


def _sin_poly(t):
    t2 = t * t
    r = jnp.float32(_SIN_C[-1])
    for c in _SIN_C[-2::-1]:
        r = r * t2 + jnp.float32(c)
    return r * t


def _cos_poly(t):
    t2 = t * t
    r = jnp.float32(_COS_C[-1])
    for c in _COS_C[-2::-1]:
        r = r * t2 + jnp.float32(c)
    return r


def _sc_gather_body(head_idx_h, rel_idx_h, tail_idx_h, ent_h, relemb_h,
                    rot_h, tail_h,
                    hidx_v, ridx_v, idx_v, head_v, rel_v, rot_v,
                    buf0, buf1, sem0, sem_a, sem_b, sem_oa, sem_ob):
    wid = lax.axis_index("s") * 2 + lax.axis_index("c")
    tb = wid * _B_PER_TILE
    row0 = tb * _NNEG

    pltpu.sync_copy(head_idx_h.at[pl.ds(tb, _B_PER_TILE)], hidx_v)
    pltpu.sync_copy(rel_idx_h.at[pl.ds(tb, _B_PER_TILE)], ridx_v)
    pltpu.sync_copy(tail_idx_h.at[pl.ds(row0, _ROWS_PER_TILE)], idx_v)

    def start_gather(c, buf, sem):
        pltpu.make_async_copy(
            ent_h.at[idx_v.at[pl.ds(pl.multiple_of(c * _CHUNK, _CHUNK),
                                    _CHUNK)]],
            buf, sem).start()

    def wait_gather(buf, sem):
        pltpu.make_async_copy(
            ent_h.at[idx_v.at[pl.ds(0, _CHUNK)]], buf, sem).wait()

    def start_out(c, buf, sem):
        pltpu.make_async_copy(
            buf, tail_h.at[pl.ds(row0 + c * _CHUNK, _CHUNK)], sem).start()

    def wait_out(buf, sem):
        pltpu.make_async_copy(
            buf, tail_h.at[pl.ds(row0, _CHUNK)], sem).wait()

    start_gather(0, buf0, sem_a)
    start_gather(1, buf1, sem_b)

    pltpu.async_copy(ent_h.at[hidx_v], head_v, sem0).wait()
    pltpu.async_copy(relemb_h.at[ridx_v], rel_v, sem0).wait()

    def rot_body(b, carry):
        for hv in range(_HIDDEN // 16):
            sl = pl.ds(hv * 16, 16)
            ph = rel_v[b, sl] * jnp.float32(_PHASE_K)
            cr = _cos_poly(ph)
            sr = _sin_poly(ph)
            rh = head_v[b, sl]
            ih = head_v[b, pl.ds(_HIDDEN + hv * 16, 16)]
            rot_v[b, sl] = rh * cr - ih * sr
            rot_v[b, pl.ds(_HIDDEN + hv * 16, 16)] = rh * sr + ih * cr
        return carry

    lax.fori_loop(0, _B_PER_TILE, rot_body, 0)
    pltpu.sync_copy(rot_v, rot_h.at[pl.ds(tb, _B_PER_TILE)])

    def pair_body(i, carry):
        wait_gather(buf0, sem_a)
        start_out(2 * i, buf0, sem_oa)
        wait_gather(buf1, sem_b)
        start_out(2 * i + 1, buf1, sem_ob)

        @pl.when(i < _PAIRS - 1)
        def _():
            wait_out(buf0, sem_oa)
            start_gather(2 * i + 2, buf0, sem_a)
            wait_out(buf1, sem_ob)
            start_gather(2 * i + 3, buf1, sem_b)

        return carry

    lax.fori_loop(0, _PAIRS, pair_body, 0)
    wait_out(buf0, sem_oa)
    wait_out(buf1, sem_ob)


@functools.lru_cache(maxsize=1)
def _build_gather():
    return functools.partial(
        pl.kernel,
        out_type=(
            jax.ShapeDtypeStruct((_B_CALL, _ENT_DIM), jnp.float32),
            jax.ShapeDtypeStruct((_CROWS, _ENT_DIM), jnp.float32),
        ),
        scratch_types=[
            pltpu.VMEM((_B_PER_TILE,), jnp.int32),
            pltpu.VMEM((_B_PER_TILE,), jnp.int32),
            pltpu.VMEM((_ROWS_PER_TILE,), jnp.int32),
            pltpu.VMEM((_B_PER_TILE, _ENT_DIM), jnp.float32),
            pltpu.VMEM((_B_PER_TILE, _HIDDEN), jnp.float32),
            pltpu.VMEM((_B_PER_TILE, _ENT_DIM), jnp.float32),
            pltpu.VMEM((_CHUNK, _ENT_DIM), jnp.float32),
            pltpu.VMEM((_CHUNK, _ENT_DIM), jnp.float32),
            pltpu.SemaphoreType.DMA,
            pltpu.SemaphoreType.DMA,
            pltpu.SemaphoreType.DMA,
            pltpu.SemaphoreType.DMA,
            pltpu.SemaphoreType.DMA,
        ],
        mesh=plsc.VectorSubcoreMesh(core_axis_name="c", subcore_axis_name="s"),
    )(_sc_gather_body)


_TC_ROWS = 8


def _tc_score_body(tail_ref, rot_ref, out_ref):
    t = tail_ref[...]
    r = rot_ref[...]
    re_t = t[:, :_HIDDEN].reshape(_TC_ROWS, _NNEG, _HIDDEN)
    im_t = t[:, _HIDDEN:].reshape(_TC_ROWS, _NNEG, _HIDDEN)
    re_r = r[:, None, :_HIDDEN]
    im_r = r[:, None, _HIDDEN:]
    d1 = re_r - re_t
    d2 = im_r - im_t
    s = jnp.sqrt(d1 * d1 + d2 * d2)
    out_ref[...] = jnp.float32(_GAMMA) - jnp.sum(s, axis=-1)


@functools.lru_cache(maxsize=1)
def _build_score():
    grid = _B_CALL // _TC_ROWS
    return pl.pallas_call(
        _tc_score_body,
        grid=(grid,),
        in_specs=[
            pl.BlockSpec((_TC_ROWS * _NNEG, _ENT_DIM), lambda i: (i, 0)),
            pl.BlockSpec((_TC_ROWS, _ENT_DIM), lambda i: (i, 0)),
        ],
        out_specs=pl.BlockSpec((_TC_ROWS, _NNEG), lambda i: (i, 0)),
        out_shape=jax.ShapeDtypeStruct((_B_CALL, _NNEG), jnp.float32),
    )


@jax.jit
def kernel(head_idx, rel_idx, neg_tail_idx, entity_embedding,
           relation_embedding):
    tail_flat = neg_tail_idx.reshape(-1)
    gather = _build_gather()
    score = _build_score()
    parts = []
    for k in range(_NCALLS):
        b0 = k * _B_CALL
        rot, tail_dense = gather(
            head_idx[b0:b0 + _B_CALL],
            rel_idx[b0:b0 + _B_CALL],
            tail_flat[b0 * _NNEG:(b0 + _B_CALL) * _NNEG],
            entity_embedding, relation_embedding)
        parts.append(score(tail_dense, rot))
    return jnp.concatenate(parts, axis=0)


# speedup vs baseline: 1.8079x; 1.0418x over previous
import functools

import jax
import jax.numpy as jnp
import numpy as np
from jax import lax
from jax.experimental import pallas as pl
from jax.experimental.pallas import tpu as pltpu
from jax.experimental.pallas import tpu_sc as plsc

_HIDDEN = 128
_ENT_DIM = 256
_B = 1024
_NNEG = 256
_GAMMA = 12.0
_EMB_RANGE = (12.0 + 2.0) / _HIDDEN
_PHASE_K = float(np.pi) / _EMB_RANGE

_TILES = 32
_NCALLS = 4
_B_CALL = _B // _NCALLS
_B_PER_TILE = _B_CALL // _TILES
_ROWS_PER_TILE = _B_PER_TILE * _NNEG
_CHUNK = 128
_PAIRS = _ROWS_PER_TILE // (2 * _CHUNK)
_CROWS = _B_CALL * _NNEG

_SIN_C = (9.9999999443e-01, -1.6666664567e-01, 8.3333102843e-03,
          -1.9840151690e-04, 2.7529392628e-06, -2.4676469125e-08,
          1.3449911084e-10)
_COS_C = (1.0000000001e+00, -4.9999999854e-01, 4.1666663479e-02,
          -1.3888863033e-03, 2.4800553772e-05, -2.7534807478e-07,
          2.0603622903e-09, -9.7225822060e-12)


def _sin_poly(t):
    t2 = t * t
    r = jnp.float32(_SIN_C[-1])
    for c in _SIN_C[-2::-1]:
        r = r * t2 + jnp.float32(c)
    return r * t


def _cos_poly(t):
    t2 = t * t
    r = jnp.float32(_COS_C[-1])
    for c in _COS_C[-2::-1]:
        r = r * t2 + jnp.float32(c)
    return r


def _sc_gather_body(head_idx_h, rel_idx_h, tail_idx_h, ent_h, relemb_h,
                    rot_h, tail_h,
                    hidx_v, ridx_v, idx_v, head_v, rel_v, rot_v,
                    buf0, buf1, sem0, sem_a, sem_b, sem_oa, sem_ob):
    wid = lax.axis_index("s") * 2 + lax.axis_index("c")
    tb = wid * _B_PER_TILE
    row0 = tb * _NNEG

    pltpu.sync_copy(head_idx_h.at[pl.ds(tb, _B_PER_TILE)], hidx_v)
    pltpu.sync_copy(rel_idx_h.at[pl.ds(tb, _B_PER_TILE)], ridx_v)
    pltpu.sync_copy(tail_idx_h.at[pl.ds(row0, _ROWS_PER_TILE)], idx_v)

    def start_gather(c, buf, sem):
        pltpu.make_async_copy(
            ent_h.at[idx_v.at[pl.ds(pl.multiple_of(c * _CHUNK, _CHUNK),
                                    _CHUNK)]],
            buf, sem).start()

    def wait_gather(buf, sem):
        pltpu.make_async_copy(
            ent_h.at[idx_v.at[pl.ds(0, _CHUNK)]], buf, sem).wait()

    def start_out(c, buf, sem):
        pltpu.make_async_copy(
            buf, tail_h.at[pl.ds(row0 + c * _CHUNK, _CHUNK)], sem).start()

    def wait_out(buf, sem):
        pltpu.make_async_copy(
            buf, tail_h.at[pl.ds(row0, _CHUNK)], sem).wait()

    start_gather(0, buf0, sem_a)
    start_gather(1, buf1, sem_b)

    pltpu.async_copy(ent_h.at[hidx_v], head_v, sem0).wait()
    pltpu.async_copy(relemb_h.at[ridx_v], rel_v, sem0).wait()

    def rot_body(b, carry):
        for hv in range(_HIDDEN // 16):
            sl = pl.ds(hv * 16, 16)
            ph = rel_v[b, sl] * jnp.float32(_PHASE_K)
            cr = _cos_poly(ph)
            sr = _sin_poly(ph)
            rh = head_v[b, sl]
            ih = head_v[b, pl.ds(_HIDDEN + hv * 16, 16)]
            rot_v[b, sl] = rh * cr - ih * sr
            rot_v[b, pl.ds(_HIDDEN + hv * 16, 16)] = rh * sr + ih * cr
        return carry

    lax.fori_loop(0, _B_PER_TILE, rot_body, 0)
    pltpu.sync_copy(rot_v, rot_h.at[pl.ds(tb, _B_PER_TILE)])

    def pair_body(i, carry):
        wait_gather(buf0, sem_a)
        start_out(2 * i, buf0, sem_oa)
        wait_gather(buf1, sem_b)
        start_out(2 * i + 1, buf1, sem_ob)

        @pl.when(i < _PAIRS - 1)
        def _():
            wait_out(buf0, sem_oa)
            start_gather(2 * i + 2, buf0, sem_a)
            wait_out(buf1, sem_ob)
            start_gather(2 * i + 3, buf1, sem_b)

        return carry

    lax.fori_loop(0, _PAIRS, pair_body, 0)
    wait_out(buf0, sem_oa)
    wait_out(buf1, sem_ob)


@functools.lru_cache(maxsize=1)
def _build_gather():
    return functools.partial(
        pl.kernel,
        out_type=(
            jax.ShapeDtypeStruct((_B_CALL, _ENT_DIM), jnp.float32),
            jax.ShapeDtypeStruct((_CROWS, _ENT_DIM), jnp.float32),
        ),
        scratch_types=[
            pltpu.VMEM((_B_PER_TILE,), jnp.int32),
            pltpu.VMEM((_B_PER_TILE,), jnp.int32),
            pltpu.VMEM((_ROWS_PER_TILE,), jnp.int32),
            pltpu.VMEM((_B_PER_TILE, _ENT_DIM), jnp.float32),
            pltpu.VMEM((_B_PER_TILE, _HIDDEN), jnp.float32),
            pltpu.VMEM((_B_PER_TILE, _ENT_DIM), jnp.float32),
            pltpu.VMEM((_CHUNK, _ENT_DIM), jnp.float32),
            pltpu.VMEM((_CHUNK, _ENT_DIM), jnp.float32),
            pltpu.SemaphoreType.DMA,
            pltpu.SemaphoreType.DMA,
            pltpu.SemaphoreType.DMA,
            pltpu.SemaphoreType.DMA,
            pltpu.SemaphoreType.DMA,
        ],
        mesh=plsc.VectorSubcoreMesh(core_axis_name="c", subcore_axis_name="s"),
    )(_sc_gather_body)


_TC_ROWS = 16


def _tc_score_body(tail_ref, rot_ref, out_ref):
    t = tail_ref[...]
    r = rot_ref[...]
    re_t = t[:, :_HIDDEN].reshape(_TC_ROWS, _NNEG, _HIDDEN)
    im_t = t[:, _HIDDEN:].reshape(_TC_ROWS, _NNEG, _HIDDEN)
    re_r = r[:, None, :_HIDDEN]
    im_r = r[:, None, _HIDDEN:]
    d1 = re_r - re_t
    d2 = im_r - im_t
    s = jnp.sqrt(d1 * d1 + d2 * d2)
    out_ref[...] = jnp.float32(_GAMMA) - jnp.sum(s, axis=-1)


@functools.lru_cache(maxsize=1)
def _build_score():
    grid = _B_CALL // _TC_ROWS
    return pl.pallas_call(
        _tc_score_body,
        grid=(grid,),
        in_specs=[
            pl.BlockSpec((_TC_ROWS * _NNEG, _ENT_DIM), lambda i: (i, 0)),
            pl.BlockSpec((_TC_ROWS, _ENT_DIM), lambda i: (i, 0)),
        ],
        out_specs=pl.BlockSpec((_TC_ROWS, _NNEG), lambda i: (i, 0)),
        out_shape=jax.ShapeDtypeStruct((_B_CALL, _NNEG), jnp.float32),
    )


@jax.jit
def kernel(head_idx, rel_idx, neg_tail_idx, entity_embedding,
           relation_embedding):
    tail_flat = neg_tail_idx.reshape(-1)
    gather = _build_gather()
    score = _build_score()
    parts = []
    for k in range(_NCALLS):
        b0 = k * _B_CALL
        rot, tail_dense = gather(
            head_idx[b0:b0 + _B_CALL],
            rel_idx[b0:b0 + _B_CALL],
            tail_flat[b0 * _NNEG:(b0 + _B_CALL) * _NNEG],
            entity_embedding, relation_embedding)
        parts.append(score(tail_dense, rot))
    return jnp.concatenate(parts, axis=0)


# speedup vs baseline: 1.8544x; 1.0257x over previous
import functools

import jax
import jax.numpy as jnp
import numpy as np
from jax import lax
from jax.experimental import pallas as pl
from jax.experimental.pallas import tpu as pltpu
from jax.experimental.pallas import tpu_sc as plsc

_HIDDEN = 128
_ENT_DIM = 256
_B = 1024
_NNEG = 256
_GAMMA = 12.0
_EMB_RANGE = (12.0 + 2.0) / _HIDDEN
_PHASE_K = float(np.pi) / _EMB_RANGE

_TILES = 32
_NCALLS = 4
_B_CALL = _B // _NCALLS
_B_PER_TILE = _B_CALL // _TILES
_ROWS_PER_TILE = _B_PER_TILE * _NNEG
_CHUNK = 128
_PAIRS = _ROWS_PER_TILE // (2 * _CHUNK)
_CROWS = _B_CALL * _NNEG

_SIN_C = (9.9999999443e-01, -1.6666664567e-01, 8.3333102843e-03,
          -1.9840151690e-04, 2.7529392628e-06, -2.4676469125e-08,
          1.3449911084e-10)
_COS_C = (1.0000000001e+00, -4.9999999854e-01, 4.1666663479e-02,
          -1.3888863033e-03, 2.4800553772e-05, -2.7534807478e-07,
          2.0603622903e-09, -9.7225822060e-12)


def _sin_poly(t):
    t2 = t * t
    r = jnp.float32(_SIN_C[-1])
    for c in _SIN_C[-2::-1]:
        r = r * t2 + jnp.float32(c)
    return r * t


def _cos_poly(t):
    t2 = t * t
    r = jnp.float32(_COS_C[-1])
    for c in _COS_C[-2::-1]:
        r = r * t2 + jnp.float32(c)
    return r


def _sc_gather_body(head_idx_h, rel_idx_h, tail_idx_h, ent_h, relemb_h,
                    rot_h, tail_h,
                    hidx_v, ridx_v, idx_v, head_v, rel_v, rot_v,
                    buf0, buf1, sem0, sem_a, sem_b, sem_oa, sem_ob):
    wid = lax.axis_index("s") * 2 + lax.axis_index("c")
    tb = wid * _B_PER_TILE
    row0 = tb * _NNEG

    pltpu.sync_copy(head_idx_h.at[pl.ds(tb, _B_PER_TILE)], hidx_v)
    pltpu.sync_copy(rel_idx_h.at[pl.ds(tb, _B_PER_TILE)], ridx_v)
    pltpu.sync_copy(tail_idx_h.at[pl.ds(row0, _ROWS_PER_TILE)], idx_v)

    def start_gather(c, buf, sem):
        pltpu.make_async_copy(
            ent_h.at[idx_v.at[pl.ds(pl.multiple_of(c * _CHUNK, _CHUNK),
                                    _CHUNK)]],
            buf, sem).start()

    def wait_gather(buf, sem):
        pltpu.make_async_copy(
            ent_h.at[idx_v.at[pl.ds(0, _CHUNK)]], buf, sem).wait()

    def start_out(c, buf, sem):
        pltpu.make_async_copy(
            buf, tail_h.at[pl.ds(row0 + c * _CHUNK, _CHUNK)], sem).start()

    def wait_out(buf, sem):
        pltpu.make_async_copy(
            buf, tail_h.at[pl.ds(row0, _CHUNK)], sem).wait()

    start_gather(0, buf0, sem_a)
    start_gather(1, buf1, sem_b)

    pltpu.async_copy(ent_h.at[hidx_v], head_v, sem0).wait()
    pltpu.async_copy(relemb_h.at[ridx_v], rel_v, sem0).wait()

    def rot_body(b, carry):
        for hv in range(_HIDDEN // 16):
            sl = pl.ds(hv * 16, 16)
            ph = rel_v[b, sl] * jnp.float32(_PHASE_K)
            cr = _cos_poly(ph)
            sr = _sin_poly(ph)
            rh = head_v[b, sl]
            ih = head_v[b, pl.ds(_HIDDEN + hv * 16, 16)]
            rot_v[b, sl] = rh * cr - ih * sr
            rot_v[b, pl.ds(_HIDDEN + hv * 16, 16)] = rh * sr + ih * cr
        return carry

    lax.fori_loop(0, _B_PER_TILE, rot_body, 0)
    pltpu.sync_copy(rot_v, rot_h.at[pl.ds(tb, _B_PER_TILE)])

    def pair_body(i, carry):
        wait_gather(buf0, sem_a)
        start_out(2 * i, buf0, sem_oa)
        wait_gather(buf1, sem_b)
        start_out(2 * i + 1, buf1, sem_ob)

        @pl.when(i < _PAIRS - 1)
        def _():
            wait_out(buf0, sem_oa)
            start_gather(2 * i + 2, buf0, sem_a)
            wait_out(buf1, sem_ob)
            start_gather(2 * i + 3, buf1, sem_b)

        return carry

    lax.fori_loop(0, _PAIRS, pair_body, 0)
    wait_out(buf0, sem_oa)
    wait_out(buf1, sem_ob)


@functools.lru_cache(maxsize=1)
def _build_gather():
    return functools.partial(
        pl.kernel,
        out_type=(
            jax.ShapeDtypeStruct((_B_CALL, _ENT_DIM), jnp.float32),
            jax.ShapeDtypeStruct((_CROWS, _ENT_DIM), jnp.float32),
        ),
        scratch_types=[
            pltpu.VMEM((_B_PER_TILE,), jnp.int32),
            pltpu.VMEM((_B_PER_TILE,), jnp.int32),
            pltpu.VMEM((_ROWS_PER_TILE,), jnp.int32),
            pltpu.VMEM((_B_PER_TILE, _ENT_DIM), jnp.float32),
            pltpu.VMEM((_B_PER_TILE, _HIDDEN), jnp.float32),
            pltpu.VMEM((_B_PER_TILE, _ENT_DIM), jnp.float32),
            pltpu.VMEM((_CHUNK, _ENT_DIM), jnp.float32),
            pltpu.VMEM((_CHUNK, _ENT_DIM), jnp.float32),
            pltpu.SemaphoreType.DMA,
            pltpu.SemaphoreType.DMA,
            pltpu.SemaphoreType.DMA,
            pltpu.SemaphoreType.DMA,
            pltpu.SemaphoreType.DMA,
        ],
        mesh=plsc.VectorSubcoreMesh(core_axis_name="c", subcore_axis_name="s"),
    )(_sc_gather_body)


_TC_ROWS = 32


def _tc_score_body(tail_ref, rot_ref, out_ref):
    t = tail_ref[...]
    r = rot_ref[...]
    re_t = t[:, :_HIDDEN].reshape(_TC_ROWS, _NNEG, _HIDDEN)
    im_t = t[:, _HIDDEN:].reshape(_TC_ROWS, _NNEG, _HIDDEN)
    re_r = r[:, None, :_HIDDEN]
    im_r = r[:, None, _HIDDEN:]
    d1 = re_r - re_t
    d2 = im_r - im_t
    s = jnp.sqrt(d1 * d1 + d2 * d2)
    out_ref[...] = jnp.float32(_GAMMA) - jnp.sum(s, axis=-1)


@functools.lru_cache(maxsize=1)
def _build_score():
    grid = _B_CALL // _TC_ROWS
    return pl.pallas_call(
        _tc_score_body,
        grid=(grid,),
        in_specs=[
            pl.BlockSpec((_TC_ROWS * _NNEG, _ENT_DIM), lambda i: (i, 0)),
            pl.BlockSpec((_TC_ROWS, _ENT_DIM), lambda i: (i, 0)),
        ],
        out_specs=pl.BlockSpec((_TC_ROWS, _NNEG), lambda i: (i, 0)),
        out_shape=jax.ShapeDtypeStruct((_B_CALL, _NNEG), jnp.float32),
    )


@jax.jit
def kernel(head_idx, rel_idx, neg_tail_idx, entity_embedding,
           relation_embedding):
    tail_flat = neg_tail_idx.reshape(-1)
    gather = _build_gather()
    score = _build_score()
    parts = []
    for k in range(_NCALLS):
        b0 = k * _B_CALL
        rot, tail_dense = gather(
            head_idx[b0:b0 + _B_CALL],
            rel_idx[b0:b0 + _B_CALL],
            tail_flat[b0 * _NNEG:(b0 + _B_CALL) * _NNEG],
            entity_embedding, relation_embedding)
        parts.append(score(tail_dense, rot))
    return jnp.concatenate(parts, axis=0)


# speedup vs baseline: 1.8833x; 1.0156x over previous
import functools

import jax
import jax.numpy as jnp
import numpy as np
from jax import lax
from jax.experimental import pallas as pl
from jax.experimental.pallas import tpu as pltpu
from jax.experimental.pallas import tpu_sc as plsc

_HIDDEN = 128
_ENT_DIM = 256
_B = 1024
_NNEG = 256
_GAMMA = 12.0
_EMB_RANGE = (12.0 + 2.0) / _HIDDEN
_PHASE_K = float(np.pi) / _EMB_RANGE

_TILES = 32
_NCALLS = 4
_B_CALL = _B // _NCALLS
_B_PER_TILE = _B_CALL // _TILES
_ROWS_PER_TILE = _B_PER_TILE * _NNEG
_CHUNK = 128
_PAIRS = _ROWS_PER_TILE // (2 * _CHUNK)
_CROWS = _B_CALL * _NNEG

_SIN_C = (9.9999999443e-01, -1.6666664567e-01, 8.3333102843e-03,
          -1.9840151690e-04, 2.7529392628e-06, -2.4676469125e-08,
          1.3449911084e-10)
_COS_C = (1.0000000001e+00, -4.9999999854e-01, 4.1666663479e-02,
          -1.3888863033e-03, 2.4800553772e-05, -2.7534807478e-07,
          2.0603622903e-09, -9.7225822060e-12)


def _sin_poly(t):
    t2 = t * t
    r = jnp.float32(_SIN_C[-1])
    for c in _SIN_C[-2::-1]:
        r = r * t2 + jnp.float32(c)
    return r * t


def _cos_poly(t):
    t2 = t * t
    r = jnp.float32(_COS_C[-1])
    for c in _COS_C[-2::-1]:
        r = r * t2 + jnp.float32(c)
    return r


def _sc_gather_body(head_idx_h, rel_idx_h, tail_idx_h, ent_h, relemb_h,
                    rot_h, tail_h,
                    hidx_v, ridx_v, idx_v, head_v, rel_v, rot_v,
                    buf0, buf1, sem0, sem_a, sem_b, sem_oa, sem_ob):
    wid = lax.axis_index("s") * 2 + lax.axis_index("c")
    tb = wid * _B_PER_TILE
    row0 = tb * _NNEG

    pltpu.sync_copy(head_idx_h.at[pl.ds(tb, _B_PER_TILE)], hidx_v)
    pltpu.sync_copy(rel_idx_h.at[pl.ds(tb, _B_PER_TILE)], ridx_v)
    pltpu.sync_copy(tail_idx_h.at[pl.ds(row0, _ROWS_PER_TILE)], idx_v)

    def start_gather(c, buf, sem):
        pltpu.make_async_copy(
            ent_h.at[idx_v.at[pl.ds(pl.multiple_of(c * _CHUNK, _CHUNK),
                                    _CHUNK)]],
            buf, sem).start()

    def wait_gather(buf, sem):
        pltpu.make_async_copy(
            ent_h.at[idx_v.at[pl.ds(0, _CHUNK)]], buf, sem).wait()

    def start_out(c, buf, sem):
        pltpu.make_async_copy(
            buf, tail_h.at[pl.ds(row0 + c * _CHUNK, _CHUNK)], sem).start()

    def wait_out(buf, sem):
        pltpu.make_async_copy(
            buf, tail_h.at[pl.ds(row0, _CHUNK)], sem).wait()

    start_gather(0, buf0, sem_a)
    start_gather(1, buf1, sem_b)

    pltpu.async_copy(ent_h.at[hidx_v], head_v, sem0).wait()
    pltpu.async_copy(relemb_h.at[ridx_v], rel_v, sem0).wait()

    def rot_body(b, carry):
        for hv in range(_HIDDEN // 16):
            sl = pl.ds(hv * 16, 16)
            ph = rel_v[b, sl] * jnp.float32(_PHASE_K)
            cr = _cos_poly(ph)
            sr = _sin_poly(ph)
            rh = head_v[b, sl]
            ih = head_v[b, pl.ds(_HIDDEN + hv * 16, 16)]
            rot_v[b, sl] = rh * cr - ih * sr
            rot_v[b, pl.ds(_HIDDEN + hv * 16, 16)] = rh * sr + ih * cr
        return carry

    lax.fori_loop(0, _B_PER_TILE, rot_body, 0)
    pltpu.sync_copy(rot_v, rot_h.at[pl.ds(tb, _B_PER_TILE)])

    def pair_body(i, carry):
        wait_gather(buf0, sem_a)
        start_out(2 * i, buf0, sem_oa)
        wait_gather(buf1, sem_b)
        start_out(2 * i + 1, buf1, sem_ob)

        @pl.when(i < _PAIRS - 1)
        def _():
            wait_out(buf0, sem_oa)
            start_gather(2 * i + 2, buf0, sem_a)
            wait_out(buf1, sem_ob)
            start_gather(2 * i + 3, buf1, sem_b)

        return carry

    lax.fori_loop(0, _PAIRS, pair_body, 0)
    wait_out(buf0, sem_oa)
    wait_out(buf1, sem_ob)


@functools.lru_cache(maxsize=1)
def _build_gather():
    return functools.partial(
        pl.kernel,
        out_type=(
            jax.ShapeDtypeStruct((_B_CALL, _ENT_DIM), jnp.float32),
            jax.ShapeDtypeStruct((_CROWS, _ENT_DIM), jnp.float32),
        ),
        scratch_types=[
            pltpu.VMEM((_B_PER_TILE,), jnp.int32),
            pltpu.VMEM((_B_PER_TILE,), jnp.int32),
            pltpu.VMEM((_ROWS_PER_TILE,), jnp.int32),
            pltpu.VMEM((_B_PER_TILE, _ENT_DIM), jnp.float32),
            pltpu.VMEM((_B_PER_TILE, _HIDDEN), jnp.float32),
            pltpu.VMEM((_B_PER_TILE, _ENT_DIM), jnp.float32),
            pltpu.VMEM((_CHUNK, _ENT_DIM), jnp.float32),
            pltpu.VMEM((_CHUNK, _ENT_DIM), jnp.float32),
            pltpu.SemaphoreType.DMA,
            pltpu.SemaphoreType.DMA,
            pltpu.SemaphoreType.DMA,
            pltpu.SemaphoreType.DMA,
            pltpu.SemaphoreType.DMA,
        ],
        mesh=plsc.VectorSubcoreMesh(core_axis_name="c", subcore_axis_name="s"),
    )(_sc_gather_body)


_TC_ROWS = 64


def _tc_score_body(tail_ref, rot_ref, out_ref):
    t = tail_ref[...]
    r = rot_ref[...]
    re_t = t[:, :_HIDDEN].reshape(_TC_ROWS, _NNEG, _HIDDEN)
    im_t = t[:, _HIDDEN:].reshape(_TC_ROWS, _NNEG, _HIDDEN)
    re_r = r[:, None, :_HIDDEN]
    im_r = r[:, None, _HIDDEN:]
    d1 = re_r - re_t
    d2 = im_r - im_t
    s = jnp.sqrt(d1 * d1 + d2 * d2)
    out_ref[...] = jnp.float32(_GAMMA) - jnp.sum(s, axis=-1)


@functools.lru_cache(maxsize=1)
def _build_score():
    grid = _B_CALL // _TC_ROWS
    return pl.pallas_call(
        _tc_score_body,
        grid=(grid,),
        in_specs=[
            pl.BlockSpec((_TC_ROWS * _NNEG, _ENT_DIM), lambda i: (i, 0)),
            pl.BlockSpec((_TC_ROWS, _ENT_DIM), lambda i: (i, 0)),
        ],
        out_specs=pl.BlockSpec((_TC_ROWS, _NNEG), lambda i: (i, 0)),
        out_shape=jax.ShapeDtypeStruct((_B_CALL, _NNEG), jnp.float32),
    )


@jax.jit
def kernel(head_idx, rel_idx, neg_tail_idx, entity_embedding,
           relation_embedding):
    tail_flat = neg_tail_idx.reshape(-1)
    gather = _build_gather()
    score = _build_score()
    parts = []
    for k in range(_NCALLS):
        b0 = k * _B_CALL
        rot, tail_dense = gather(
            head_idx[b0:b0 + _B_CALL],
            rel_idx[b0:b0 + _B_CALL],
            tail_flat[b0 * _NNEG:(b0 + _B_CALL) * _NNEG],
            entity_embedding, relation_embedding)
        parts.append(score(tail_dense, rot))
    return jnp.concatenate(parts, axis=0)
